# Initial kernel scaffold; baseline (speedup 1.0000x reference)
import jax
import jax.numpy as jnp
from jax.experimental import pallas as pl


def kernel(x1, edge_index1, edge_attr1, batch1, x2, edge_index2, edge_attr2, W1, b1, W2, b2, fc1_w, fc1_b, ln1_g, ln1_b, fc2_w, fc2_b, ln2_g, ln2_b, fc3_w, fc3_b):
    raise NotImplementedError("write your pallas kernel here")


# speedup vs baseline: 1.0611x; 1.0611x over previous
import functools
import jax
import jax.numpy as jnp
from jax import lax
from jax.experimental import pallas as pl
from jax.experimental.pallas import tpu as pltpu

N1 = 10000; N2 = 199; F = 128; B = 16; K = 50
N2P = 208


def _cdist_body(a_ref, b_ref, bsq_ref, d_ref, vals_ref):
    a = a_ref[...]
    b = b_ref[...]
    ab = lax.dot_general(a, b, (((1,), (1,)), ((), ())),
                         preferred_element_type=jnp.float32)
    asq = jnp.sum(a * a, axis=1, keepdims=True)
    d2 = asq + bsq_ref[...] - 2.0 * ab
    d = jnp.sqrt(jnp.clip(d2, 1e-12, None))
    d_ref[...] = d
    vals_ref[...] = d[:, N2 - 1:N2]


def _cdist(o1, o2p):
    bsq = jnp.sum(o2p * o2p, axis=1)[None, :]
    blk = 400
    grid = (N1 // blk,)
    return pl.pallas_call(
        _cdist_body,
        grid=grid,
        in_specs=[
            pl.BlockSpec((blk, 64), lambda i: (i, 0)),
            pl.BlockSpec((N2P, 64), lambda i: (0, 0)),
            pl.BlockSpec((1, N2P), lambda i: (0, 0)),
        ],
        out_specs=[
            pl.BlockSpec((blk, N2P), lambda i: (i, 0)),
            pl.BlockSpec((blk, 1), lambda i: (i, 0)),
        ],
        out_shape=[
            jax.ShapeDtypeStruct((N1, N2P), jnp.float32),
            jax.ShapeDtypeStruct((N1, 1), jnp.float32),
        ],
    )(o1, o2p, bsq)


def kernel(x1, edge_index1, edge_attr1, batch1, x2, edge_index2, edge_attr2,
           W1, b1, W2, b2, fc1_w, fc1_b, ln1_g, ln1_b, fc2_w, fc2_b,
           ln2_g, ln2_b, fc3_w, fc3_b):
    Nc = N1 + N2
    x = jnp.concatenate([x1, x2])
    src = jnp.concatenate([edge_index1[0], edge_index2[0] + N1])
    dst = jnp.concatenate([edge_index1[1], edge_index2[1] + N1])
    w = jnp.concatenate([edge_attr1, edge_attr2])
    deg = jnp.zeros((Nc,), jnp.float32).at[dst].add(w) + 1.0
    dinv = deg ** -0.5
    norm = dinv[src] * w * dinv[dst]

    def layer(h_in, W, b):
        h = h_in @ W
        out = jnp.zeros_like(h).at[dst].add(norm[:, None] * h[src])
        out = out + dinv[:, None] ** 2 * h + b
        return jax.nn.relu(out)

    h1 = layer(x, W1, b1)
    h2 = layer(h1, W2, b2)
    o1, o2 = h2[:N1], h2[N1:]
    o2p = jnp.concatenate([o2, jnp.zeros((N2P - N2, 64), jnp.float32)])
    d, vals = _cdist(o1, o2p)
    vals = vals[:, 0]

    M = jnp.where(batch1[None, :] == jnp.arange(B)[:, None], vals[None, :], -jnp.inf)
    tv, ti = jax.lax.top_k(M, K)
    mask = jnp.isfinite(tv)
    rows = d[ti][:, :, :N2] * mask[:, :, None]
    agg = rows.reshape(B, -1)

    def layer_norm(x, g, b):
        m = x.mean(-1, keepdims=True)
        v = ((x - m) ** 2).mean(-1, keepdims=True)
        return (x - m) / jnp.sqrt(v + 1e-5) * g + b

    h = agg @ fc1_w + fc1_b
    h = jax.nn.relu(layer_norm(h, ln1_g, ln1_b))
    h = h @ fc2_w + fc2_b
    h = jax.nn.relu(layer_norm(h, ln2_g, ln2_b))
    h = h @ fc3_w + fc3_b
    return jax.nn.sigmoid(h)


# speedup vs baseline: 11.1341x; 10.4929x over previous
import functools
import jax
import jax.numpy as jnp
from jax import lax
from jax.experimental import pallas as pl
from jax.experimental.pallas import tpu as pltpu, tpu_sc as plsc

N1 = 10000; N2 = 199; F = 128; B = 16; K = 50
N2P = 208
NC = N1 + N2
NCP = 10240
E = 320000 + 3184
EP = 327680
CH = 128
SC_C, SC_S = 2, 16
NW = SC_C * SC_S
EW = EP // NW
NCHUNK = EW // CH
RPS = NCP // SC_S


def _sc_mesh():
    return plsc.VectorSubcoreMesh(core_axis_name="c", subcore_axis_name="s")


@functools.cache
def _make_deg_kernel():
    @functools.partial(
        pl.kernel,
        out_type=jax.ShapeDtypeStruct((2, NCP), jnp.float32),
        mesh=_sc_mesh(),
        scratch_types=[
            pltpu.VMEM((CH,), jnp.int32),
            pltpu.VMEM((CH,), jnp.float32),
            pltpu.VMEM_SHARED((NCP,), jnp.float32),
        ],
    )
    def deg_kernel(dst_hbm, w_hbm, zero_hbm, out_hbm, idx_d, w_v, acc):
        c = lax.axis_index("c")
        s = lax.axis_index("s")
        wid = s * SC_C + c
        pltpu.sync_copy(zero_hbm.at[pl.ds(s * RPS, RPS)],
                        acc.at[pl.ds(s * RPS, RPS)])
        plsc.subcore_barrier()

        def chunk(gi, _):
            base = wid * EW + gi * CH
            pltpu.sync_copy(dst_hbm.at[pl.ds(base, CH)], idx_d)
            pltpu.sync_copy(w_hbm.at[pl.ds(base, CH)], w_v)
            pltpu.sync_copy(w_v, acc.at[idx_d], add=True)
            return 0

        lax.fori_loop(0, NCHUNK, chunk, 0)
        plsc.subcore_barrier()
        pltpu.sync_copy(acc.at[pl.ds(s * RPS, RPS)],
                        out_hbm.at[c, pl.ds(s * RPS, RPS)])

    return deg_kernel


@functools.cache
def _make_agg_kernel(f):
    @functools.partial(
        pl.kernel,
        out_type=jax.ShapeDtypeStruct((2, NCP, f), jnp.float32),
        mesh=_sc_mesh(),
        scratch_types=[
            pltpu.VMEM((CH,), jnp.int32),
            pltpu.VMEM((CH,), jnp.int32),
            pltpu.VMEM((CH,), jnp.float32),
            pltpu.VMEM((CH, f), jnp.float32),
            pltpu.VMEM_SHARED((NCP, f), jnp.float32),
            pltpu.SemaphoreType.DMA,
        ],
    )
    def agg_kernel(src_hbm, dst_hbm, w_hbm, g_hbm, zero_hbm, out_hbm,
                   idx_s, idx_d, w_v, rows, acc, sem):
        c = lax.axis_index("c")
        s = lax.axis_index("s")
        wid = s * SC_C + c
        pltpu.sync_copy(zero_hbm.at[pl.ds(s * RPS, RPS)],
                        acc.at[pl.ds(s * RPS, RPS)])
        plsc.subcore_barrier()

        def chunk(gi, _):
            base = wid * EW + gi * CH
            pltpu.sync_copy(src_hbm.at[pl.ds(base, CH)], idx_s)
            pltpu.sync_copy(dst_hbm.at[pl.ds(base, CH)], idx_d)
            pltpu.sync_copy(w_hbm.at[pl.ds(base, CH)], w_v)
            pltpu.async_copy(g_hbm.at[idx_s], rows, sem).wait()

            def scale(g2, _):
                w16 = w_v[pl.ds(g2 * 16, 16)]
                for l in range(16):
                    nv = w16[l]
                    e = g2 * 16 + l
                    for k in range(f // 16):
                        rows[e, pl.ds(16 * k, 16)] = rows[e, pl.ds(16 * k, 16)] * nv
                return 0

            lax.fori_loop(0, CH // 16, scale, 0)
            pltpu.sync_copy(rows, acc.at[idx_d], add=True)
            return 0

        lax.fori_loop(0, NCHUNK, chunk, 0)
        plsc.subcore_barrier()
        pltpu.sync_copy(acc.at[pl.ds(s * RPS, RPS)],
                        out_hbm.at[c, pl.ds(s * RPS, RPS)])

    return agg_kernel


def _cdist_body(a_ref, b_ref, bsq_ref, d_ref, vals_ref):
    a = a_ref[...]
    b = b_ref[...]
    ab = lax.dot_general(a, b, (((1,), (1,)), ((), ())),
                         preferred_element_type=jnp.float32)
    asq = jnp.sum(a * a, axis=1, keepdims=True)
    d2 = asq + bsq_ref[...] - 2.0 * ab
    d = jnp.sqrt(jnp.clip(d2, 1e-12, None))
    d_ref[...] = d
    vals_ref[...] = d[:, N2 - 1:N2]


def _cdist(o1, o2p):
    bsq = jnp.sum(o2p * o2p, axis=1)[None, :]
    blk = 400
    return pl.pallas_call(
        _cdist_body,
        grid=(N1 // blk,),
        in_specs=[
            pl.BlockSpec((blk, 64), lambda i: (i, 0)),
            pl.BlockSpec((N2P, 64), lambda i: (0, 0)),
            pl.BlockSpec((1, N2P), lambda i: (0, 0)),
        ],
        out_specs=[
            pl.BlockSpec((blk, N2P), lambda i: (i, 0)),
            pl.BlockSpec((blk, 1), lambda i: (i, 0)),
        ],
        out_shape=[
            jax.ShapeDtypeStruct((N1, N2P), jnp.float32),
            jax.ShapeDtypeStruct((N1, 1), jnp.float32),
        ],
    )(o1, o2p, bsq)


def kernel(x1, edge_index1, edge_attr1, batch1, x2, edge_index2, edge_attr2,
           W1, b1, W2, b2, fc1_w, fc1_b, ln1_g, ln1_b, fc2_w, fc2_b,
           ln2_g, ln2_b, fc3_w, fc3_b):
    pad_idx = (jnp.arange(EP - E, dtype=jnp.int32) % NCP)
    src = jnp.concatenate([edge_index1[0].astype(jnp.int32),
                           edge_index2[0].astype(jnp.int32) + N1, pad_idx])
    dst = jnp.concatenate([edge_index1[1].astype(jnp.int32),
                           edge_index2[1].astype(jnp.int32) + N1, pad_idx])
    w = jnp.concatenate([edge_attr1, edge_attr2,
                         jnp.zeros((EP - E,), jnp.float32)])

    xp = jnp.concatenate([x1, x2, jnp.zeros((NCP - NC, F), jnp.float32)])

    degp = _make_deg_kernel()(dst, w, jnp.zeros((NCP,), jnp.float32))
    deg = degp[0] + degp[1] + 1.0
    dinv = deg ** -0.5
    dinv2 = dinv * dinv

    def layer(h_in, W, b, f):
        h = h_in @ W
        g = dinv[:, None] * h
        p = _make_agg_kernel(f)(src, dst, w, g, jnp.zeros((NCP, f), jnp.float32))
        out = dinv[:, None] * (p[0] + p[1]) + dinv2[:, None] * h + b
        return jax.nn.relu(out)

    h1 = layer(xp, W1, b1, 128)
    W2p = jnp.concatenate([W2, jnp.zeros((128, 64), jnp.float32)], axis=1)
    b2p = jnp.concatenate([b2, jnp.zeros((64,), jnp.float32)])
    h2 = layer(h1, W2p, b2p, 128)[:, :64]
    o1, o2 = h2[:N1], h2[N1:NC]
    o2p = jnp.concatenate([o2, jnp.zeros((N2P - N2, 64), jnp.float32)])
    d, vals = _cdist(o1, o2p)
    vals = vals[:, 0]

    M = jnp.where(batch1[None, :] == jnp.arange(B)[:, None], vals[None, :], -jnp.inf)
    tv, ti = jax.lax.top_k(M, K)
    mask = jnp.isfinite(tv)
    rows = d[ti][:, :, :N2] * mask[:, :, None]
    agg = rows.reshape(B, -1)

    def layer_norm(x, g, b):
        m = x.mean(-1, keepdims=True)
        v = ((x - m) ** 2).mean(-1, keepdims=True)
        return (x - m) / jnp.sqrt(v + 1e-5) * g + b

    h = agg @ fc1_w + fc1_b
    h = jax.nn.relu(layer_norm(h, ln1_g, ln1_b))
    h = h @ fc2_w + fc2_b
    h = jax.nn.relu(layer_norm(h, ln2_g, ln2_b))
    h = h @ fc3_w + fc3_b
    return jax.nn.sigmoid(h)


# speedup vs baseline: 12.8470x; 1.1539x over previous
import functools
import jax
import jax.numpy as jnp
from jax import lax
from jax.experimental import pallas as pl
from jax.experimental.pallas import tpu as pltpu, tpu_sc as plsc

N1 = 10000; N2 = 199; F = 128; B = 16; K = 50
N2P = 208
NC = N1 + N2
NCP = 10240
E = 320000 + 3184
EP = 327680
SC_C, SC_S = 2, 16
NW = SC_C * SC_S
EW = EP // NW
RPS = NCP // SC_S
NBUF = 4


def _sc_mesh():
    return plsc.VectorSubcoreMesh(core_axis_name="c", subcore_axis_name="s")


@functools.cache
def _make_deg_kernel():
    CH = 128
    NCHUNK = EW // CH

    @functools.partial(
        pl.kernel,
        out_type=jax.ShapeDtypeStruct((2, NCP), jnp.float32),
        mesh=_sc_mesh(),
        scratch_types=(
            [pltpu.VMEM((CH,), jnp.int32) for _ in range(NBUF)] +
            [pltpu.VMEM((CH,), jnp.float32) for _ in range(NBUF)] +
            [pltpu.SemaphoreType.DMA for _ in range(NBUF)] +
            [pltpu.VMEM_SHARED((NCP,), jnp.float32)]
        ),
    )
    def deg_kernel(dst_hbm, w_hbm, zero_hbm, out_hbm, *scr):
        idx_d = scr[0:NBUF]
        w_v = scr[NBUF:2 * NBUF]
        ssem = scr[2 * NBUF:3 * NBUF]
        acc = scr[3 * NBUF]
        c = lax.axis_index("c")
        s = lax.axis_index("s")
        wid = s * SC_C + c
        wbase = wid * EW
        pltpu.sync_copy(zero_hbm.at[pl.ds(s * RPS, RPS)],
                        acc.at[pl.ds(s * RPS, RPS)])
        plsc.subcore_barrier()

        def load(k, b):
            base = wbase + k * CH
            pltpu.sync_copy(dst_hbm.at[pl.ds(base, CH)], idx_d[b])
            pltpu.sync_copy(w_hbm.at[pl.ds(base, CH)], w_v[b])

        for b in range(NBUF - 1):
            load(b, b)

        def body(j, _):
            for p in range(NBUF):
                i = j * NBUF + p
                bk = (p - 1) % NBUF
                k = i + NBUF - 1
                pltpu.async_copy(w_v[p], acc.at[idx_d[p]], ssem[p], add=True)

                @pl.when(k < NCHUNK)
                def _():
                    @pl.when(i >= 1)
                    def _():
                        pltpu.make_async_copy(w_v[bk], acc.at[idx_d[bk]],
                                              ssem[bk]).wait()
                    load(k, bk)
            return 0

        lax.fori_loop(0, NCHUNK // NBUF, body, 0)
        for b in range(NBUF):
            pltpu.make_async_copy(w_v[b], acc.at[idx_d[b]], ssem[b]).wait()
        plsc.subcore_barrier()
        pltpu.sync_copy(acc.at[pl.ds(s * RPS, RPS)],
                        out_hbm.at[c, pl.ds(s * RPS, RPS)])

    return deg_kernel


@functools.cache
def _make_agg_kernel(f):
    CH = 64
    NCHUNK = EW // CH

    @functools.partial(
        pl.kernel,
        out_type=jax.ShapeDtypeStruct((2, NCP, f), jnp.float32),
        mesh=_sc_mesh(),
        scratch_types=(
            [pltpu.VMEM((CH,), jnp.int32) for _ in range(NBUF)] +
            [pltpu.VMEM((CH,), jnp.int32) for _ in range(NBUF)] +
            [pltpu.VMEM((CH,), jnp.float32) for _ in range(NBUF)] +
            [pltpu.VMEM((CH, f), jnp.float32) for _ in range(NBUF)] +
            [pltpu.SemaphoreType.DMA for _ in range(2 * NBUF)] +
            [pltpu.VMEM_SHARED((NCP, f), jnp.float32)]
        ),
    )
    def agg_kernel(src_hbm, dst_hbm, w_hbm, g_hbm, zero_hbm, out_hbm, *scr):
        idx_s = scr[0:NBUF]
        idx_d = scr[NBUF:2 * NBUF]
        w_v = scr[2 * NBUF:3 * NBUF]
        rows = scr[3 * NBUF:4 * NBUF]
        gsem = scr[4 * NBUF:5 * NBUF]
        ssem = scr[5 * NBUF:6 * NBUF]
        acc = scr[6 * NBUF]
        c = lax.axis_index("c")
        s = lax.axis_index("s")
        wid = s * SC_C + c
        wbase = wid * EW

        pltpu.sync_copy(zero_hbm.at[pl.ds(s * RPS, RPS)],
                        acc.at[pl.ds(s * RPS, RPS)])
        plsc.subcore_barrier()

        def load_and_fire(k, b):
            base = wbase + k * CH
            pltpu.sync_copy(src_hbm.at[pl.ds(base, CH)], idx_s[b])
            pltpu.sync_copy(dst_hbm.at[pl.ds(base, CH)], idx_d[b])
            pltpu.sync_copy(w_hbm.at[pl.ds(base, CH)], w_v[b])
            pltpu.async_copy(g_hbm.at[idx_s[b]], rows[b], gsem[b])

        for b in range(NBUF - 1):
            load_and_fire(b, b)

        def scale(b):
            def grp(g2, _):
                w16 = w_v[b][pl.ds(g2 * 16, 16)]
                for l in range(16):
                    nv = w16[l]
                    e = g2 * 16 + l
                    for k in range(f // 16):
                        rows[b][e, pl.ds(16 * k, 16)] = (
                            rows[b][e, pl.ds(16 * k, 16)] * nv)
                return 0
            lax.fori_loop(0, CH // 16, grp, 0)

        def body(j, _):
            for p in range(NBUF):
                i = j * NBUF + p
                bk = (p - 1) % NBUF
                k = i + NBUF - 1
                pltpu.make_async_copy(g_hbm.at[idx_s[p]], rows[p], gsem[p]).wait()
                scale(p)
                pltpu.async_copy(rows[p], acc.at[idx_d[p]], ssem[p], add=True)

                @pl.when(k < NCHUNK)
                def _():
                    @pl.when(i >= 1)
                    def _():
                        pltpu.make_async_copy(rows[bk], acc.at[idx_d[bk]],
                                              ssem[bk]).wait()
                    load_and_fire(k, bk)
            return 0

        lax.fori_loop(0, NCHUNK // NBUF, body, 0)
        for b in range(NBUF):
            pltpu.make_async_copy(rows[b], acc.at[idx_d[b]], ssem[b]).wait()
        plsc.subcore_barrier()
        pltpu.sync_copy(acc.at[pl.ds(s * RPS, RPS)],
                        out_hbm.at[c, pl.ds(s * RPS, RPS)])

    return agg_kernel


def _cdist_body(a_ref, b_ref, bsq_ref, d_ref, vals_ref):
    a = a_ref[...]
    b = b_ref[...]
    ab = lax.dot_general(a, b, (((1,), (1,)), ((), ())),
                         preferred_element_type=jnp.float32)
    asq = jnp.sum(a * a, axis=1, keepdims=True)
    d2 = asq + bsq_ref[...] - 2.0 * ab
    d = jnp.sqrt(jnp.clip(d2, 1e-12, None))
    d_ref[...] = d
    vals_ref[...] = d[:, N2 - 1:N2]


def _cdist(o1, o2p):
    bsq = jnp.sum(o2p * o2p, axis=1)[None, :]
    blk = 400
    return pl.pallas_call(
        _cdist_body,
        grid=(N1 // blk,),
        in_specs=[
            pl.BlockSpec((blk, 64), lambda i: (i, 0)),
            pl.BlockSpec((N2P, 64), lambda i: (0, 0)),
            pl.BlockSpec((1, N2P), lambda i: (0, 0)),
        ],
        out_specs=[
            pl.BlockSpec((blk, N2P), lambda i: (i, 0)),
            pl.BlockSpec((blk, 1), lambda i: (i, 0)),
        ],
        out_shape=[
            jax.ShapeDtypeStruct((N1, N2P), jnp.float32),
            jax.ShapeDtypeStruct((N1, 1), jnp.float32),
        ],
    )(o1, o2p, bsq)


def kernel(x1, edge_index1, edge_attr1, batch1, x2, edge_index2, edge_attr2,
           W1, b1, W2, b2, fc1_w, fc1_b, ln1_g, ln1_b, fc2_w, fc2_b,
           ln2_g, ln2_b, fc3_w, fc3_b):
    pad_idx = (jnp.arange(EP - E, dtype=jnp.int32) % NCP)
    src = jnp.concatenate([edge_index1[0].astype(jnp.int32),
                           edge_index2[0].astype(jnp.int32) + N1, pad_idx])
    dst = jnp.concatenate([edge_index1[1].astype(jnp.int32),
                           edge_index2[1].astype(jnp.int32) + N1, pad_idx])
    w = jnp.concatenate([edge_attr1, edge_attr2,
                         jnp.zeros((EP - E,), jnp.float32)])

    xp = jnp.concatenate([x1, x2, jnp.zeros((NCP - NC, F), jnp.float32)])

    degp = _make_deg_kernel()(dst, w, jnp.zeros((NCP,), jnp.float32))
    deg = degp[0] + degp[1] + 1.0
    dinv = deg ** -0.5
    dinv2 = dinv * dinv

    def layer(h_in, W, b, f):
        h = h_in @ W
        g = dinv[:, None] * h
        p = _make_agg_kernel(f)(src, dst, w, g, jnp.zeros((NCP, f), jnp.float32))
        out = dinv[:, None] * (p[0] + p[1]) + dinv2[:, None] * h + b
        return jax.nn.relu(out)

    h1 = layer(xp, W1, b1, 128)
    W2p = jnp.concatenate([W2, jnp.zeros((128, 64), jnp.float32)], axis=1)
    b2p = jnp.concatenate([b2, jnp.zeros((64,), jnp.float32)])
    h2 = layer(h1, W2p, b2p, 128)[:, :64]
    o1, o2 = h2[:N1], h2[N1:NC]
    o2p = jnp.concatenate([o2, jnp.zeros((N2P - N2, 64), jnp.float32)])
    d, vals = _cdist(o1, o2p)
    vals = vals[:, 0]

    M = jnp.where(batch1[None, :] == jnp.arange(B)[:, None], vals[None, :], -jnp.inf)
    tv, ti = jax.lax.top_k(M, K)
    mask = jnp.isfinite(tv)
    rows = d[ti][:, :, :N2] * mask[:, :, None]
    agg = rows.reshape(B, -1)

    def layer_norm(x, g, b):
        m = x.mean(-1, keepdims=True)
        v = ((x - m) ** 2).mean(-1, keepdims=True)
        return (x - m) / jnp.sqrt(v + 1e-5) * g + b

    h = agg @ fc1_w + fc1_b
    h = jax.nn.relu(layer_norm(h, ln1_g, ln1_b))
    h = h @ fc2_w + fc2_b
    h = jax.nn.relu(layer_norm(h, ln2_g, ln2_b))
    h = h @ fc3_w + fc3_b
    return jax.nn.sigmoid(h)


# speedup vs baseline: 17.8618x; 1.3903x over previous
import functools
import jax
import jax.numpy as jnp
from jax import lax
from jax.experimental import pallas as pl
from jax.experimental.pallas import tpu as pltpu, tpu_sc as plsc

N1 = 10000; N2 = 199; F = 128; B = 16; K = 50
N2P = 208
NC = N1 + N2
NCP = 10240
E = 320000 + 3184
EP = 327680
SC_C, SC_S = 2, 16
NW = SC_C * SC_S
EW = EP // NW
RPS = NCP // SC_S
NBUF = 4


def _sc_mesh():
    return plsc.VectorSubcoreMesh(core_axis_name="c", subcore_axis_name="s")


@functools.cache
def _make_deg_kernel():
    CH = 128
    NCHUNK = EW // CH

    @functools.partial(
        pl.kernel,
        out_type=jax.ShapeDtypeStruct((2, NCP), jnp.float32),
        mesh=_sc_mesh(),
        scratch_types=(
            [pltpu.VMEM((CH,), jnp.int32) for _ in range(NBUF)] +
            [pltpu.VMEM((CH,), jnp.float32) for _ in range(NBUF)] +
            [pltpu.SemaphoreType.DMA for _ in range(NBUF)] +
            [pltpu.VMEM_SHARED((NCP,), jnp.float32)]
        ),
    )
    def deg_kernel(dst_hbm, w_hbm, zero_hbm, out_hbm, *scr):
        idx_d = scr[0:NBUF]
        w_v = scr[NBUF:2 * NBUF]
        ssem = scr[2 * NBUF:3 * NBUF]
        acc = scr[3 * NBUF]
        c = lax.axis_index("c")
        s = lax.axis_index("s")
        wid = s * SC_C + c
        wbase = wid * EW
        pltpu.sync_copy(zero_hbm.at[pl.ds(s * RPS, RPS)],
                        acc.at[pl.ds(s * RPS, RPS)])
        plsc.subcore_barrier()

        def load(k, b):
            base = wbase + k * CH
            pltpu.sync_copy(dst_hbm.at[pl.ds(base, CH)], idx_d[b])
            pltpu.sync_copy(w_hbm.at[pl.ds(base, CH)], w_v[b])

        for b in range(NBUF - 1):
            load(b, b)

        def body(j, _):
            for p in range(NBUF):
                i = j * NBUF + p
                bk = (p - 1) % NBUF
                k = i + NBUF - 1
                pltpu.async_copy(w_v[p], acc.at[idx_d[p]], ssem[p], add=True)

                @pl.when(k < NCHUNK)
                def _():
                    @pl.when(i >= 1)
                    def _():
                        pltpu.make_async_copy(w_v[bk], acc.at[idx_d[bk]],
                                              ssem[bk]).wait()
                    load(k, bk)
            return 0

        lax.fori_loop(0, NCHUNK // NBUF, body, 0)
        for b in range(NBUF):
            pltpu.make_async_copy(w_v[b], acc.at[idx_d[b]], ssem[b]).wait()
        plsc.subcore_barrier()
        pltpu.sync_copy(acc.at[pl.ds(s * RPS, RPS)],
                        out_hbm.at[c, pl.ds(s * RPS, RPS)])

    return deg_kernel


@functools.cache
def _make_agg_kernel(f):
    CH = 64
    NCHUNK = EW // CH

    @functools.partial(
        pl.kernel,
        out_type=jax.ShapeDtypeStruct((2, NCP, f), jnp.float32),
        mesh=_sc_mesh(),
        scratch_types=(
            [pltpu.VMEM((3, CH), jnp.int32) for _ in range(NBUF)] +
            [pltpu.VMEM((CH, f), jnp.float32) for _ in range(NBUF)] +
            [pltpu.SemaphoreType.DMA for _ in range(2 * NBUF)] +
            [pltpu.VMEM_SHARED((NCP, f), jnp.float32)]
        ),
    )
    def agg_kernel(p_hbm, g_hbm, zero_hbm, out_hbm, *scr):
        sdw = scr[0:NBUF]
        rows = scr[NBUF:2 * NBUF]
        gsem = scr[2 * NBUF:3 * NBUF]
        ssem = scr[3 * NBUF:4 * NBUF]
        acc = scr[4 * NBUF]
        c = lax.axis_index("c")
        s = lax.axis_index("s")
        wid = s * SC_C + c
        cbase = wid * NCHUNK

        pltpu.sync_copy(zero_hbm.at[pl.ds(s * RPS, RPS)],
                        acc.at[pl.ds(s * RPS, RPS)])
        plsc.subcore_barrier()

        def load_and_fire(k, b):
            pltpu.sync_copy(p_hbm.at[cbase + k], sdw[b])
            pltpu.async_copy(g_hbm.at[sdw[b].at[0]], rows[b], gsem[b])

        for b in range(NBUF - 1):
            load_and_fire(b, b)

        def scale(b):
            def grp(g2, _):
                w16 = lax.bitcast_convert_type(sdw[b][2, pl.ds(g2 * 16, 16)],
                                               jnp.float32)
                for l in range(16):
                    nv = w16[l]
                    e = g2 * 16 + l
                    for k in range(f // 16):
                        rows[b][e, pl.ds(16 * k, 16)] = (
                            rows[b][e, pl.ds(16 * k, 16)] * nv)
                return 0
            lax.fori_loop(0, CH // 16, grp, 0)

        def body(j, _):
            for p in range(NBUF):
                i = j * NBUF + p
                bk = (p - 1) % NBUF
                k = i + NBUF - 1
                pltpu.make_async_copy(g_hbm.at[sdw[p].at[0]], rows[p],
                                      gsem[p]).wait()
                scale(p)
                pltpu.async_copy(rows[p], acc.at[sdw[p].at[1]], ssem[p],
                                 add=True)

                @pl.when(k < NCHUNK)
                def _():
                    @pl.when(i >= 1)
                    def _():
                        pltpu.make_async_copy(rows[bk], acc.at[sdw[bk].at[1]],
                                              ssem[bk]).wait()
                    load_and_fire(k, bk)
            return 0

        lax.fori_loop(0, NCHUNK // NBUF, body, 0)
        for b in range(NBUF):
            pltpu.make_async_copy(rows[b], acc.at[sdw[b].at[1]], ssem[b]).wait()
        plsc.subcore_barrier()
        pltpu.sync_copy(acc.at[pl.ds(s * RPS, RPS)],
                        out_hbm.at[c, pl.ds(s * RPS, RPS)])

    return agg_kernel


def _cdist_body(a_ref, b_ref, bsq_ref, d_ref, vals_ref):
    a = a_ref[...]
    b = b_ref[...]
    ab = lax.dot_general(a, b, (((1,), (1,)), ((), ())),
                         preferred_element_type=jnp.float32)
    asq = jnp.sum(a * a, axis=1, keepdims=True)
    d2 = asq + bsq_ref[...] - 2.0 * ab
    d = jnp.sqrt(jnp.clip(d2, 1e-12, None))
    d_ref[...] = d
    vals_ref[...] = d[:, N2 - 1:N2]


def _cdist(o1, o2p):
    bsq = jnp.sum(o2p * o2p, axis=1)[None, :]
    blk = 400
    return pl.pallas_call(
        _cdist_body,
        grid=(N1 // blk,),
        in_specs=[
            pl.BlockSpec((blk, 64), lambda i: (i, 0)),
            pl.BlockSpec((N2P, 64), lambda i: (0, 0)),
            pl.BlockSpec((1, N2P), lambda i: (0, 0)),
        ],
        out_specs=[
            pl.BlockSpec((blk, N2P), lambda i: (i, 0)),
            pl.BlockSpec((blk, 1), lambda i: (i, 0)),
        ],
        out_shape=[
            jax.ShapeDtypeStruct((N1, N2P), jnp.float32),
            jax.ShapeDtypeStruct((N1, 1), jnp.float32),
        ],
    )(o1, o2p, bsq)


def kernel(x1, edge_index1, edge_attr1, batch1, x2, edge_index2, edge_attr2,
           W1, b1, W2, b2, fc1_w, fc1_b, ln1_g, ln1_b, fc2_w, fc2_b,
           ln2_g, ln2_b, fc3_w, fc3_b):
    pad_idx = (jnp.arange(EP - E, dtype=jnp.int32) % NCP)
    src = jnp.concatenate([edge_index1[0].astype(jnp.int32),
                           edge_index2[0].astype(jnp.int32) + N1, pad_idx])
    dst = jnp.concatenate([edge_index1[1].astype(jnp.int32),
                           edge_index2[1].astype(jnp.int32) + N1, pad_idx])
    w = jnp.concatenate([edge_attr1, edge_attr2,
                         jnp.zeros((EP - E,), jnp.float32)])

    xp = jnp.concatenate([x1, x2, jnp.zeros((NCP - NC, F), jnp.float32)])

    degp = _make_deg_kernel()(dst, w, jnp.zeros((NCP,), jnp.float32))
    deg = degp[0] + degp[1] + 1.0
    dinv = deg ** -0.5
    dinv2 = dinv * dinv

    CH = 64
    packed = jnp.stack([src.reshape(-1, CH), dst.reshape(-1, CH),
                        lax.bitcast_convert_type(w, jnp.int32).reshape(-1, CH)],
                       axis=1)

    def layer(h_in, W, b, f):
        h = h_in @ W
        g = dinv[:, None] * h
        p = _make_agg_kernel(f)(packed, g, jnp.zeros((NCP, f), jnp.float32))
        out = dinv[:, None] * (p[0] + p[1]) + dinv2[:, None] * h + b
        return jax.nn.relu(out)

    h1 = layer(xp, W1, b1, 128)
    W2p = jnp.concatenate([W2, jnp.zeros((128, 64), jnp.float32)], axis=1)
    b2p = jnp.concatenate([b2, jnp.zeros((64,), jnp.float32)])
    h2 = layer(h1, W2p, b2p, 128)[:, :64]
    o1, o2 = h2[:N1], h2[N1:NC]
    o2p = jnp.concatenate([o2, jnp.zeros((N2P - N2, 64), jnp.float32)])
    d, vals = _cdist(o1, o2p)
    vals = vals[:, 0]

    M = jnp.where(batch1[None, :] == jnp.arange(B)[:, None], vals[None, :], -jnp.inf)
    tv, ti = jax.lax.top_k(M, K)
    mask = jnp.isfinite(tv)
    rows = d[ti][:, :, :N2] * mask[:, :, None]
    agg = rows.reshape(B, -1)

    def layer_norm(x, g, b):
        m = x.mean(-1, keepdims=True)
        v = ((x - m) ** 2).mean(-1, keepdims=True)
        return (x - m) / jnp.sqrt(v + 1e-5) * g + b

    h = agg @ fc1_w + fc1_b
    h = jax.nn.relu(layer_norm(h, ln1_g, ln1_b))
    h = h @ fc2_w + fc2_b
    h = jax.nn.relu(layer_norm(h, ln2_g, ln2_b))
    h = h @ fc3_w + fc3_b
    return jax.nn.sigmoid(h)


# speedup vs baseline: 20.7965x; 1.1643x over previous
import functools
import jax
import jax.numpy as jnp
from jax import lax
from jax.experimental import pallas as pl
from jax.experimental.pallas import tpu as pltpu, tpu_sc as plsc

N1 = 10000; N2 = 199; F = 128; B = 16; K = 50
N2P = 208
NC = N1 + N2
NCP = 10240
E = 320000 + 3184
EP = 327680
SC_C, SC_S = 2, 16
NW = SC_C * SC_S
EW = EP // NW
RPS = NCP // SC_S
NBUF = 4


def _sc_mesh():
    return plsc.VectorSubcoreMesh(core_axis_name="c", subcore_axis_name="s")


@functools.cache
def _make_deg_kernel():
    CH = 128
    NCHUNK = EW // CH

    @functools.partial(
        pl.kernel,
        out_type=jax.ShapeDtypeStruct((2, NCP), jnp.float32),
        mesh=_sc_mesh(),
        scratch_types=(
            [pltpu.VMEM((2, CH), jnp.int32) for _ in range(NBUF)] +
            [pltpu.VMEM((CH,), jnp.float32) for _ in range(NBUF)] +
            [pltpu.SemaphoreType.DMA for _ in range(2 * NBUF)] +
            [pltpu.VMEM_SHARED((NCP,), jnp.float32)]
        ),
    )
    def deg_kernel(p_hbm, zero_hbm, out_hbm, *scr):
        dw = scr[0:NBUF]
        w_v = scr[NBUF:2 * NBUF]
        ssem = scr[2 * NBUF:3 * NBUF]
        isem = scr[3 * NBUF:4 * NBUF]
        acc = scr[4 * NBUF]
        c = lax.axis_index("c")
        s = lax.axis_index("s")
        wid = s * SC_C + c
        cbase = wid * NCHUNK
        pltpu.sync_copy(zero_hbm.at[pl.ds(s * RPS, RPS)],
                        acc.at[pl.ds(s * RPS, RPS)])
        plsc.subcore_barrier()

        def fire_load(k, b):
            pltpu.async_copy(p_hbm.at[cbase + k], dw[b], isem[b])

        def wait_load(b):
            pltpu.make_async_copy(p_hbm.at[cbase], dw[b], isem[b]).wait()

        def conv(b):
            for g2 in range(CH // 16):
                w_v[b][pl.ds(g2 * 16, 16)] = lax.bitcast_convert_type(
                    dw[b][1, pl.ds(g2 * 16, 16)], jnp.float32)

        for b in range(NBUF - 1):
            fire_load(b, b)

        def body(j, _):
            for p in range(NBUF):
                i = j * NBUF + p
                bk = (p - 1) % NBUF
                k = i + NBUF - 1
                wait_load(p)
                conv(p)
                pltpu.async_copy(w_v[p], acc.at[dw[p].at[0]], ssem[p], add=True)

                @pl.when(k < NCHUNK)
                def _():
                    @pl.when(i >= 1)
                    def _():
                        pltpu.make_async_copy(w_v[bk], acc.at[dw[bk].at[0]],
                                              ssem[bk]).wait()
                    fire_load(k, bk)
            return 0

        lax.fori_loop(0, NCHUNK // NBUF, body, 0)
        for b in range(NBUF):
            pltpu.make_async_copy(w_v[b], acc.at[dw[b].at[0]], ssem[b]).wait()
        plsc.subcore_barrier()
        pltpu.sync_copy(acc.at[pl.ds(s * RPS, RPS)],
                        out_hbm.at[c, pl.ds(s * RPS, RPS)])

    return deg_kernel


@functools.cache
def _make_agg_kernel(f):
    CH = 64
    NCHUNK = EW // CH

    @functools.partial(
        pl.kernel,
        out_type=jax.ShapeDtypeStruct((2, NCP, f), jnp.float32),
        mesh=_sc_mesh(),
        scratch_types=(
            [pltpu.VMEM((3, CH), jnp.int32) for _ in range(NBUF)] +
            [pltpu.VMEM((CH, f), jnp.float32) for _ in range(NBUF)] +
            [pltpu.SemaphoreType.DMA for _ in range(3 * NBUF)] +
            [pltpu.VMEM_SHARED((NCP, f), jnp.float32)]
        ),
    )
    def agg_kernel(p_hbm, g_hbm, zero_hbm, out_hbm, *scr):
        sdw = scr[0:NBUF]
        rows = scr[NBUF:2 * NBUF]
        gsem = scr[2 * NBUF:3 * NBUF]
        ssem = scr[3 * NBUF:4 * NBUF]
        isem = scr[4 * NBUF:5 * NBUF]
        acc = scr[5 * NBUF]
        c = lax.axis_index("c")
        s = lax.axis_index("s")
        wid = s * SC_C + c
        cbase = wid * NCHUNK

        pltpu.sync_copy(zero_hbm.at[pl.ds(s * RPS, RPS)],
                        acc.at[pl.ds(s * RPS, RPS)])
        plsc.subcore_barrier()

        def fire_load(k, b):
            pltpu.async_copy(p_hbm.at[cbase + k], sdw[b], isem[b])

        def wait_load(b):
            pltpu.make_async_copy(p_hbm.at[cbase], sdw[b], isem[b]).wait()

        def fire_gather(b):
            pltpu.async_copy(g_hbm.at[sdw[b].at[0]], rows[b], gsem[b])

        def wait_gather(b):
            pltpu.make_async_copy(g_hbm.at[sdw[b].at[0]], rows[b],
                                  gsem[b]).wait()

        for b in range(3):
            fire_load(b, b)
        for b in range(2):
            wait_load(b)
            fire_gather(b)

        def scale(b):
            def grp(g2, _):
                w16 = lax.bitcast_convert_type(sdw[b][2, pl.ds(g2 * 16, 16)],
                                               jnp.float32)
                for l in range(16):
                    nv = w16[l]
                    e = g2 * 16 + l
                    for k in range(f // 16):
                        rows[b][e, pl.ds(16 * k, 16)] = (
                            rows[b][e, pl.ds(16 * k, 16)] * nv)
                return 0
            lax.fori_loop(0, CH // 16, grp, 0)

        def body(j, _):
            for p in range(NBUF):
                i = j * NBUF + p
                bk = (p - 1) % NBUF
                bg = (p + 2) % NBUF
                k = i + NBUF - 1
                wait_gather(p)
                scale(p)
                pltpu.async_copy(rows[p], acc.at[sdw[p].at[1]], ssem[p],
                                 add=True)

                @pl.when(k < NCHUNK)
                def _():
                    @pl.when(i >= 1)
                    def _():
                        pltpu.make_async_copy(rows[bk], acc.at[sdw[bk].at[1]],
                                              ssem[bk]).wait()
                    fire_load(k, bk)

                @pl.when(i + 2 < NCHUNK)
                def _():
                    wait_load(bg)
                    fire_gather(bg)
            return 0

        lax.fori_loop(0, NCHUNK // NBUF, body, 0)
        for b in range(NBUF):
            pltpu.make_async_copy(rows[b], acc.at[sdw[b].at[1]], ssem[b]).wait()
        plsc.subcore_barrier()
        pltpu.sync_copy(acc.at[pl.ds(s * RPS, RPS)],
                        out_hbm.at[c, pl.ds(s * RPS, RPS)])

    return agg_kernel


def _cdist_body(a_ref, b_ref, bsq_ref, d_ref, vals_ref):
    a = a_ref[...]
    b = b_ref[...]
    ab = lax.dot_general(a, b, (((1,), (1,)), ((), ())),
                         preferred_element_type=jnp.float32)
    asq = jnp.sum(a * a, axis=1, keepdims=True)
    d2 = asq + bsq_ref[...] - 2.0 * ab
    d = jnp.sqrt(jnp.clip(d2, 1e-12, None))
    d_ref[...] = d
    vals_ref[...] = d[:, N2 - 1:N2]


def _cdist(o1, o2p):
    bsq = jnp.sum(o2p * o2p, axis=1)[None, :]
    blk = 400
    return pl.pallas_call(
        _cdist_body,
        grid=(N1 // blk,),
        in_specs=[
            pl.BlockSpec((blk, 64), lambda i: (i, 0)),
            pl.BlockSpec((N2P, 64), lambda i: (0, 0)),
            pl.BlockSpec((1, N2P), lambda i: (0, 0)),
        ],
        out_specs=[
            pl.BlockSpec((blk, N2P), lambda i: (i, 0)),
            pl.BlockSpec((blk, 1), lambda i: (i, 0)),
        ],
        out_shape=[
            jax.ShapeDtypeStruct((N1, N2P), jnp.float32),
            jax.ShapeDtypeStruct((N1, 1), jnp.float32),
        ],
    )(o1, o2p, bsq)


def kernel(x1, edge_index1, edge_attr1, batch1, x2, edge_index2, edge_attr2,
           W1, b1, W2, b2, fc1_w, fc1_b, ln1_g, ln1_b, fc2_w, fc2_b,
           ln2_g, ln2_b, fc3_w, fc3_b):
    pad_idx = (jnp.arange(EP - E, dtype=jnp.int32) % NCP)
    src = jnp.concatenate([edge_index1[0].astype(jnp.int32),
                           edge_index2[0].astype(jnp.int32) + N1, pad_idx])
    dst = jnp.concatenate([edge_index1[1].astype(jnp.int32),
                           edge_index2[1].astype(jnp.int32) + N1, pad_idx])
    w = jnp.concatenate([edge_attr1, edge_attr2,
                         jnp.zeros((EP - E,), jnp.float32)])

    xp = jnp.concatenate([x1, x2, jnp.zeros((NCP - NC, F), jnp.float32)])

    wbits = lax.bitcast_convert_type(w, jnp.int32)
    pdeg = jnp.stack([dst.reshape(-1, 128), wbits.reshape(-1, 128)], axis=1)
    degp = _make_deg_kernel()(pdeg, jnp.zeros((NCP,), jnp.float32))
    deg = degp[0] + degp[1] + 1.0
    dinv = deg ** -0.5
    dinv2 = dinv * dinv

    CH = 64
    packed = jnp.stack([src.reshape(-1, CH), dst.reshape(-1, CH),
                        wbits.reshape(-1, CH)], axis=1)

    def layer(h_in, W, b, f):
        h = h_in @ W
        g = dinv[:, None] * h
        p = _make_agg_kernel(f)(packed, g, jnp.zeros((NCP, f), jnp.float32))
        out = dinv[:, None] * (p[0] + p[1]) + dinv2[:, None] * h + b
        return jax.nn.relu(out)

    h1 = layer(xp, W1, b1, 128)
    W2p = jnp.concatenate([W2, jnp.zeros((128, 64), jnp.float32)], axis=1)
    b2p = jnp.concatenate([b2, jnp.zeros((64,), jnp.float32)])
    h2 = layer(h1, W2p, b2p, 128)[:, :64]
    o1, o2 = h2[:N1], h2[N1:NC]
    o2p = jnp.concatenate([o2, jnp.zeros((N2P - N2, 64), jnp.float32)])
    d, vals = _cdist(o1, o2p)
    vals = vals[:, 0]

    M = jnp.where(batch1[None, :] == jnp.arange(B)[:, None], vals[None, :], -jnp.inf)
    tv, ti = jax.lax.top_k(M, K)
    mask = jnp.isfinite(tv)
    rows = d[ti][:, :, :N2] * mask[:, :, None]
    agg = rows.reshape(B, -1)

    def layer_norm(x, g, b):
        m = x.mean(-1, keepdims=True)
        v = ((x - m) ** 2).mean(-1, keepdims=True)
        return (x - m) / jnp.sqrt(v + 1e-5) * g + b

    h = agg @ fc1_w + fc1_b
    h = jax.nn.relu(layer_norm(h, ln1_g, ln1_b))
    h = h @ fc2_w + fc2_b
    h = jax.nn.relu(layer_norm(h, ln2_g, ln2_b))
    h = h @ fc3_w + fc3_b
    return jax.nn.sigmoid(h)


# speedup vs baseline: 22.0437x; 1.0600x over previous
import functools
import jax
import jax.numpy as jnp
from jax import lax
from jax.experimental import pallas as pl
from jax.experimental.pallas import tpu as pltpu, tpu_sc as plsc

N1 = 10000; N2 = 199; F = 128; B = 16; K = 50
N2P = 208
NC = N1 + N2
NCP = 10240
E = 320000 + 3184
EP = 327680
SC_C, SC_S = 2, 16
NW = SC_C * SC_S
EW = EP // NW
RPS = NCP // SC_S
NBUF = 4


def _sc_mesh():
    return plsc.VectorSubcoreMesh(core_axis_name="c", subcore_axis_name="s")


@functools.cache
def _make_deg_kernel():
    CH = 128
    NCHUNK = EW // CH

    @functools.partial(
        pl.kernel,
        out_type=jax.ShapeDtypeStruct((2, NCP), jnp.float32),
        mesh=_sc_mesh(),
        scratch_types=(
            [pltpu.VMEM((2, CH), jnp.int32) for _ in range(NBUF)] +
            [pltpu.VMEM((CH,), jnp.float32) for _ in range(NBUF)] +
            [pltpu.SemaphoreType.DMA for _ in range(2 * NBUF)] +
            [pltpu.VMEM_SHARED((NCP,), jnp.float32)]
        ),
    )
    def deg_kernel(p_hbm, zero_hbm, out_hbm, *scr):
        dw = scr[0:NBUF]
        w_v = scr[NBUF:2 * NBUF]
        ssem = scr[2 * NBUF:3 * NBUF]
        isem = scr[3 * NBUF:4 * NBUF]
        acc = scr[4 * NBUF]
        c = lax.axis_index("c")
        s = lax.axis_index("s")
        wid = s * SC_C + c
        cbase = wid * NCHUNK
        pltpu.sync_copy(zero_hbm.at[pl.ds(s * RPS, RPS)],
                        acc.at[pl.ds(s * RPS, RPS)])
        plsc.subcore_barrier()

        def fire_load(k, b):
            pltpu.async_copy(p_hbm.at[cbase + k], dw[b], isem[b])

        def wait_load(b):
            pltpu.make_async_copy(p_hbm.at[cbase], dw[b], isem[b]).wait()

        def conv(b):
            for g2 in range(CH // 16):
                w_v[b][pl.ds(g2 * 16, 16)] = lax.bitcast_convert_type(
                    dw[b][1, pl.ds(g2 * 16, 16)], jnp.float32)

        for b in range(NBUF - 1):
            fire_load(b, b)

        def body(j, _):
            for p in range(NBUF):
                i = j * NBUF + p
                bk = (p - 1) % NBUF
                k = i + NBUF - 1
                wait_load(p)
                conv(p)
                pltpu.async_copy(w_v[p], acc.at[dw[p].at[0]], ssem[p], add=True)

                @pl.when(k < NCHUNK)
                def _():
                    @pl.when(i >= 1)
                    def _():
                        pltpu.make_async_copy(w_v[bk], acc.at[dw[bk].at[0]],
                                              ssem[bk]).wait()
                    fire_load(k, bk)
            return 0

        lax.fori_loop(0, NCHUNK // NBUF, body, 0)
        for b in range(NBUF):
            pltpu.make_async_copy(w_v[b], acc.at[dw[b].at[0]], ssem[b]).wait()
        plsc.subcore_barrier()
        pltpu.sync_copy(acc.at[pl.ds(s * RPS, RPS)],
                        out_hbm.at[c, pl.ds(s * RPS, RPS)])

    return deg_kernel


@functools.cache
def _make_agg_kernel(f):
    CH = 64
    NCHUNK = EW // CH

    @functools.partial(
        pl.kernel,
        out_type=jax.ShapeDtypeStruct((2, NCP, f), jnp.float32),
        mesh=_sc_mesh(),
        scratch_types=(
            [pltpu.VMEM((3, CH), jnp.int32) for _ in range(NBUF)] +
            [pltpu.VMEM((CH, f), jnp.float32) for _ in range(NBUF)] +
            [pltpu.SemaphoreType.DMA for _ in range(3 * NBUF)] +
            [pltpu.VMEM_SHARED((NCP, f), jnp.float32)]
        ),
    )
    def agg_kernel(p_hbm, g_hbm, zero_hbm, out_hbm, *scr):
        sdw = scr[0:NBUF]
        rows = scr[NBUF:2 * NBUF]
        gsem = scr[2 * NBUF:3 * NBUF]
        ssem = scr[3 * NBUF:4 * NBUF]
        isem = scr[4 * NBUF:5 * NBUF]
        acc = scr[5 * NBUF]
        c = lax.axis_index("c")
        s = lax.axis_index("s")
        wid = s * SC_C + c
        cbase = wid * NCHUNK

        pltpu.sync_copy(zero_hbm.at[pl.ds(s * RPS, RPS)],
                        acc.at[pl.ds(s * RPS, RPS)])
        plsc.subcore_barrier()

        def fire_load(k, b):
            pltpu.async_copy(p_hbm.at[cbase + k], sdw[b], isem[b])

        def wait_load(b):
            pltpu.make_async_copy(p_hbm.at[cbase], sdw[b], isem[b]).wait()

        def fire_gather(b):
            pltpu.async_copy(g_hbm.at[sdw[b].at[0]], rows[b], gsem[b])

        def wait_gather(b):
            pltpu.make_async_copy(g_hbm.at[sdw[b].at[0]], rows[b],
                                  gsem[b]).wait()

        for b in range(3):
            fire_load(b, b)
        for b in range(2):
            wait_load(b)
            fire_gather(b)

        def scale(b):
            def grp(g2, _):
                w16 = lax.bitcast_convert_type(sdw[b][2, pl.ds(g2 * 16, 16)],
                                               jnp.float32)
                for l in range(16):
                    nv = w16[l]
                    e = g2 * 16 + l
                    for k in range(f // 16):
                        rows[b][e, pl.ds(16 * k, 16)] = (
                            rows[b][e, pl.ds(16 * k, 16)] * nv)
                return 0
            lax.fori_loop(0, CH // 16, grp, 0)

        def body(j, _):
            for p in range(NBUF):
                i = j * NBUF + p
                bk = (p - 1) % NBUF
                bg = (p + 2) % NBUF
                k = i + NBUF - 1
                wait_gather(p)
                scale(p)
                pltpu.async_copy(rows[p], acc.at[sdw[p].at[1]], ssem[p],
                                 add=True)

                @pl.when(k < NCHUNK)
                def _():
                    @pl.when(i >= 1)
                    def _():
                        pltpu.make_async_copy(rows[bk], acc.at[sdw[bk].at[1]],
                                              ssem[bk]).wait()
                    fire_load(k, bk)

                @pl.when(i + 2 < NCHUNK)
                def _():
                    wait_load(bg)
                    fire_gather(bg)
            return 0

        lax.fori_loop(0, NCHUNK // NBUF, body, 0)
        for b in range(NBUF):
            pltpu.make_async_copy(rows[b], acc.at[sdw[b].at[1]], ssem[b]).wait()
        plsc.subcore_barrier()
        pltpu.sync_copy(acc.at[pl.ds(s * RPS, RPS)],
                        out_hbm.at[c, pl.ds(s * RPS, RPS)])

    return agg_kernel


def _vals_body(a_ref, b_ref, vals_ref):
    a = a_ref[...]
    b = b_ref[...]
    ab = lax.dot_general(a, b, (((1,), (1,)), ((), ())),
                         preferred_element_type=jnp.float32)[:, 0:1]
    asq = jnp.sum(a * a, axis=1, keepdims=True)
    bsq = jnp.sum(b[0:1] * b[0:1], axis=1)
    d2 = asq + bsq[None, :] - 2.0 * ab
    vals_ref[...] = jnp.sqrt(jnp.clip(d2, 1e-12, None))


def _vals(o1, blast):
    blk = 400
    return pl.pallas_call(
        _vals_body,
        grid=(N1 // blk,),
        in_specs=[
            pl.BlockSpec((blk, 64), lambda i: (i, 0)),
            pl.BlockSpec((8, 64), lambda i: (0, 0)),
        ],
        out_specs=pl.BlockSpec((blk, 1), lambda i: (i, 0)),
        out_shape=jax.ShapeDtypeStruct((N1, 1), jnp.float32),
    )(o1, blast)


def _dsel_body(a_ref, b_ref, bsq_ref, m_ref, d_ref):
    a = a_ref[...]
    b = b_ref[...]
    ab = lax.dot_general(a, b, (((1,), (1,)), ((), ())),
                         preferred_element_type=jnp.float32)
    asq = jnp.sum(a * a, axis=1, keepdims=True)
    d2 = asq + bsq_ref[...] - 2.0 * ab
    d_ref[...] = jnp.sqrt(jnp.clip(d2, 1e-12, None)) * m_ref[...]


def _dsel(o1sel, o2p, rowmask):
    bsq = jnp.sum(o2p * o2p, axis=1)[None, :]
    return pl.pallas_call(
        _dsel_body,
        out_shape=jax.ShapeDtypeStruct((B * K, N2P), jnp.float32),
    )(o1sel, o2p, bsq, rowmask)


def _head_body(x_ref, w1_ref, b1_ref, g1_ref, bb1_ref, w2_ref, b2_ref,
               g2_ref, bb2_ref, w3_ref, b3_ref, o_ref):
    def layer_norm(x, g, b):
        m = x.mean(-1, keepdims=True)
        v = ((x - m) ** 2).mean(-1, keepdims=True)
        return (x - m) / jnp.sqrt(v + 1e-5) * g + b

    h = jnp.dot(x_ref[...], w1_ref[...],
                preferred_element_type=jnp.float32) + b1_ref[...]
    h = jax.nn.relu(layer_norm(h, g1_ref[...], bb1_ref[...]))
    h = jnp.dot(h, w2_ref[...], preferred_element_type=jnp.float32) + b2_ref[...]
    h = jax.nn.relu(layer_norm(h, g2_ref[...], bb2_ref[...]))
    h = jnp.dot(h, w3_ref[...], preferred_element_type=jnp.float32) + b3_ref[...]
    o_ref[...] = jax.nn.sigmoid(h)


def _head(agg2, fc1p, fc1_b, ln1_g, ln1_b, fc2_w, fc2_b, ln2_g, ln2_b,
          fc3_w, fc3_b):
    return pl.pallas_call(
        _head_body,
        out_shape=jax.ShapeDtypeStruct((B, 1), jnp.float32),
    )(agg2, fc1p, fc1_b[None, :], ln1_g[None, :], ln1_b[None, :],
      fc2_w, fc2_b[None, :], ln2_g[None, :], ln2_b[None, :],
      fc3_w, fc3_b[None, :])


def kernel(x1, edge_index1, edge_attr1, batch1, x2, edge_index2, edge_attr2,
           W1, b1, W2, b2, fc1_w, fc1_b, ln1_g, ln1_b, fc2_w, fc2_b,
           ln2_g, ln2_b, fc3_w, fc3_b):
    pad_idx = (jnp.arange(EP - E, dtype=jnp.int32) % NCP)
    src = jnp.concatenate([edge_index1[0].astype(jnp.int32),
                           edge_index2[0].astype(jnp.int32) + N1, pad_idx])
    dst = jnp.concatenate([edge_index1[1].astype(jnp.int32),
                           edge_index2[1].astype(jnp.int32) + N1, pad_idx])
    w = jnp.concatenate([edge_attr1, edge_attr2,
                         jnp.zeros((EP - E,), jnp.float32)])

    xp = jnp.concatenate([x1, x2, jnp.zeros((NCP - NC, F), jnp.float32)])

    wbits = lax.bitcast_convert_type(w, jnp.int32)
    pdeg = jnp.stack([dst.reshape(-1, 128), wbits.reshape(-1, 128)], axis=1)
    degp = _make_deg_kernel()(pdeg, jnp.zeros((NCP,), jnp.float32))
    deg = degp[0] + degp[1] + 1.0
    dinv = deg ** -0.5
    dinv2 = dinv * dinv

    CH = 64
    packed = jnp.stack([src.reshape(-1, CH), dst.reshape(-1, CH),
                        wbits.reshape(-1, CH)], axis=1)

    def layer(h_in, W, b, f):
        h = h_in @ W
        g = dinv[:, None] * h
        p = _make_agg_kernel(f)(packed, g, jnp.zeros((NCP, f), jnp.float32))
        out = dinv[:, None] * (p[0] + p[1]) + dinv2[:, None] * h + b
        return jax.nn.relu(out)

    h1 = layer(xp, W1, b1, 128)
    W2p = jnp.concatenate([W2, jnp.zeros((128, 64), jnp.float32)], axis=1)
    b2p = jnp.concatenate([b2, jnp.zeros((64,), jnp.float32)])
    h2 = layer(h1, W2p, b2p, 128)[:, :64]
    o1, o2 = h2[:N1], h2[N1:NC]
    o2p = jnp.concatenate([o2, jnp.zeros((N2P - N2, 64), jnp.float32)])
    blast = jnp.concatenate([o2[N2 - 1:N2], jnp.zeros((7, 64), jnp.float32)])
    vals = _vals(o1, blast)[:, 0]

    M = jnp.where(batch1[None, :] == jnp.arange(B)[:, None], vals[None, :], -jnp.inf)
    tv, ti = jax.lax.top_k(M, K)
    mask = jnp.isfinite(tv)
    o1sel = o1[ti.reshape(B * K)]
    rowmask = mask.reshape(B * K, 1).astype(jnp.float32)
    dsel = _dsel(o1sel, o2p, rowmask)
    agg2 = dsel.reshape(B, K * N2P)

    fc1p = jnp.concatenate(
        [fc1_w.reshape(K, N2, 128),
         jnp.zeros((K, N2P - N2, 128), jnp.float32)], axis=1
    ).reshape(K * N2P, 128)
    return _head(agg2, fc1p, fc1_b, ln1_g, ln1_b, fc2_w, fc2_b,
                 ln2_g, ln2_b, fc3_w, fc3_b)


# speedup vs baseline: 22.7576x; 1.0324x over previous
import functools
import jax
import jax.numpy as jnp
from jax import lax
from jax.experimental import pallas as pl
from jax.experimental.pallas import tpu as pltpu, tpu_sc as plsc

N1 = 10000; N2 = 199; F = 128; B = 16; K = 50
N2P = 208
NC = N1 + N2
NCP = 10240
E = 320000 + 3184
EP = 327680
SC_C, SC_S = 2, 16
NW = SC_C * SC_S
EW = EP // NW
RPS = NCP // SC_S
NBUF = 4


def _sc_mesh():
    return plsc.VectorSubcoreMesh(core_axis_name="c", subcore_axis_name="s")


@functools.cache
def _make_deg_kernel():
    CH = 128
    NCHUNK = EW // CH

    @functools.partial(
        pl.kernel,
        out_type=jax.ShapeDtypeStruct((2, NCP), jnp.float32),
        mesh=_sc_mesh(),
        scratch_types=(
            [pltpu.VMEM((2, CH), jnp.int32) for _ in range(NBUF)] +
            [pltpu.VMEM((CH,), jnp.float32) for _ in range(NBUF)] +
            [pltpu.SemaphoreType.DMA for _ in range(2 * NBUF)] +
            [pltpu.VMEM_SHARED((NCP,), jnp.float32)]
        ),
    )
    def deg_kernel(p_hbm, zero_hbm, out_hbm, *scr):
        dw = scr[0:NBUF]
        w_v = scr[NBUF:2 * NBUF]
        ssem = scr[2 * NBUF:3 * NBUF]
        isem = scr[3 * NBUF:4 * NBUF]
        acc = scr[4 * NBUF]
        c = lax.axis_index("c")
        s = lax.axis_index("s")
        wid = s * SC_C + c
        cbase = wid * NCHUNK
        pltpu.sync_copy(zero_hbm.at[pl.ds(s * RPS, RPS)],
                        acc.at[pl.ds(s * RPS, RPS)])
        plsc.subcore_barrier()

        def fire_load(k, b):
            pltpu.async_copy(p_hbm.at[cbase + k], dw[b], isem[b])

        def wait_load(b):
            pltpu.make_async_copy(p_hbm.at[cbase], dw[b], isem[b]).wait()

        def conv(b):
            for g2 in range(CH // 16):
                w_v[b][pl.ds(g2 * 16, 16)] = lax.bitcast_convert_type(
                    dw[b][1, pl.ds(g2 * 16, 16)], jnp.float32)

        for b in range(NBUF - 1):
            fire_load(b, b)

        def body(j, _):
            for p in range(NBUF):
                i = j * NBUF + p
                bk = (p - 1) % NBUF
                k = i + NBUF - 1
                wait_load(p)
                conv(p)
                pltpu.async_copy(w_v[p], acc.at[dw[p].at[0]], ssem[p], add=True)

                @pl.when(k < NCHUNK)
                def _():
                    @pl.when(i >= 1)
                    def _():
                        pltpu.make_async_copy(w_v[bk], acc.at[dw[bk].at[0]],
                                              ssem[bk]).wait()
                    fire_load(k, bk)
            return 0

        lax.fori_loop(0, NCHUNK // NBUF, body, 0)
        for b in range(NBUF):
            pltpu.make_async_copy(w_v[b], acc.at[dw[b].at[0]], ssem[b]).wait()
        plsc.subcore_barrier()
        pltpu.sync_copy(acc.at[pl.ds(s * RPS, RPS)],
                        out_hbm.at[c, pl.ds(s * RPS, RPS)])

    return deg_kernel


@functools.cache
def _make_agg_kernel(f):
    CH = 64
    NCHUNK = EW // CH

    @functools.partial(
        pl.kernel,
        out_type=jax.ShapeDtypeStruct((2, NCP, f), jnp.float32),
        mesh=_sc_mesh(),
        scratch_types=(
            [pltpu.VMEM((3, CH), jnp.int32) for _ in range(NBUF)] +
            [pltpu.VMEM((CH, f), jnp.float32) for _ in range(NBUF)] +
            [pltpu.SemaphoreType.DMA for _ in range(3 * NBUF)] +
            [pltpu.VMEM_SHARED((NCP, f), jnp.float32)]
        ),
        compiler_params=pltpu.CompilerParams(use_tc_tiling_on_sc=False)
        if f == 64 else None,
    )
    def agg_kernel(p_hbm, g_hbm, zero_hbm, out_hbm, *scr):
        sdw = scr[0:NBUF]
        rows = scr[NBUF:2 * NBUF]
        gsem = scr[2 * NBUF:3 * NBUF]
        ssem = scr[3 * NBUF:4 * NBUF]
        isem = scr[4 * NBUF:5 * NBUF]
        acc = scr[5 * NBUF]
        c = lax.axis_index("c")
        s = lax.axis_index("s")
        wid = s * SC_C + c
        cbase = wid * NCHUNK

        pltpu.sync_copy(zero_hbm.at[pl.ds(s * RPS, RPS)],
                        acc.at[pl.ds(s * RPS, RPS)])
        plsc.subcore_barrier()

        def fire_load(k, b):
            pltpu.async_copy(p_hbm.at[cbase + k], sdw[b], isem[b])

        def wait_load(b):
            pltpu.make_async_copy(p_hbm.at[cbase], sdw[b], isem[b]).wait()

        def fire_gather(b):
            pltpu.async_copy(g_hbm.at[sdw[b].at[0]], rows[b], gsem[b])

        def wait_gather(b):
            pltpu.make_async_copy(g_hbm.at[sdw[b].at[0]], rows[b],
                                  gsem[b]).wait()

        for b in range(3):
            fire_load(b, b)
        for b in range(2):
            wait_load(b)
            fire_gather(b)

        def scale(b):
            def grp(g2, _):
                w16 = lax.bitcast_convert_type(sdw[b][2, pl.ds(g2 * 16, 16)],
                                               jnp.float32)
                for l in range(16):
                    nv = w16[l]
                    e = g2 * 16 + l
                    for k in range(f // 16):
                        rows[b][e, pl.ds(16 * k, 16)] = (
                            rows[b][e, pl.ds(16 * k, 16)] * nv)
                return 0
            lax.fori_loop(0, CH // 16, grp, 0)

        def body(j, _):
            for p in range(NBUF):
                i = j * NBUF + p
                bk = (p - 1) % NBUF
                bg = (p + 2) % NBUF
                k = i + NBUF - 1
                wait_gather(p)
                scale(p)
                pltpu.async_copy(rows[p], acc.at[sdw[p].at[1]], ssem[p],
                                 add=True)

                @pl.when(k < NCHUNK)
                def _():
                    @pl.when(i >= 1)
                    def _():
                        pltpu.make_async_copy(rows[bk], acc.at[sdw[bk].at[1]],
                                              ssem[bk]).wait()
                    fire_load(k, bk)

                @pl.when(i + 2 < NCHUNK)
                def _():
                    wait_load(bg)
                    fire_gather(bg)
            return 0

        lax.fori_loop(0, NCHUNK // NBUF, body, 0)
        for b in range(NBUF):
            pltpu.make_async_copy(rows[b], acc.at[sdw[b].at[1]], ssem[b]).wait()
        plsc.subcore_barrier()
        pltpu.sync_copy(acc.at[pl.ds(s * RPS, RPS)],
                        out_hbm.at[c, pl.ds(s * RPS, RPS)])

    return agg_kernel


def _vals_body(a_ref, b_ref, vals_ref):
    a = a_ref[...]
    b = b_ref[...]
    ab = lax.dot_general(a, b, (((1,), (1,)), ((), ())),
                         preferred_element_type=jnp.float32)[:, 0:1]
    asq = jnp.sum(a * a, axis=1, keepdims=True)
    bsq = jnp.sum(b[0:1] * b[0:1], axis=1)
    d2 = asq + bsq[None, :] - 2.0 * ab
    vals_ref[...] = jnp.sqrt(jnp.clip(d2, 1e-12, None))


def _vals(o1, blast):
    blk = 400
    return pl.pallas_call(
        _vals_body,
        grid=(N1 // blk,),
        in_specs=[
            pl.BlockSpec((blk, 64), lambda i: (i, 0)),
            pl.BlockSpec((8, 64), lambda i: (0, 0)),
        ],
        out_specs=pl.BlockSpec((blk, 1), lambda i: (i, 0)),
        out_shape=jax.ShapeDtypeStruct((N1, 1), jnp.float32),
    )(o1, blast)


def _dsel_body(a_ref, b_ref, bsq_ref, m_ref, d_ref):
    a = a_ref[...]
    b = b_ref[...]
    ab = lax.dot_general(a, b, (((1,), (1,)), ((), ())),
                         preferred_element_type=jnp.float32)
    asq = jnp.sum(a * a, axis=1, keepdims=True)
    d2 = asq + bsq_ref[...] - 2.0 * ab
    d_ref[...] = jnp.sqrt(jnp.clip(d2, 1e-12, None)) * m_ref[...]


def _dsel(o1sel, o2p, rowmask):
    bsq = jnp.sum(o2p * o2p, axis=1)[None, :]
    return pl.pallas_call(
        _dsel_body,
        out_shape=jax.ShapeDtypeStruct((B * K, N2P), jnp.float32),
    )(o1sel, o2p, bsq, rowmask)


def _head_body(x_ref, w1_ref, b1_ref, g1_ref, bb1_ref, w2_ref, b2_ref,
               g2_ref, bb2_ref, w3_ref, b3_ref, o_ref):
    def layer_norm(x, g, b):
        m = x.mean(-1, keepdims=True)
        v = ((x - m) ** 2).mean(-1, keepdims=True)
        return (x - m) / jnp.sqrt(v + 1e-5) * g + b

    h = jnp.dot(x_ref[...], w1_ref[...],
                preferred_element_type=jnp.float32) + b1_ref[...]
    h = jax.nn.relu(layer_norm(h, g1_ref[...], bb1_ref[...]))
    h = jnp.dot(h, w2_ref[...], preferred_element_type=jnp.float32) + b2_ref[...]
    h = jax.nn.relu(layer_norm(h, g2_ref[...], bb2_ref[...]))
    h = jnp.dot(h, w3_ref[...], preferred_element_type=jnp.float32) + b3_ref[...]
    o_ref[...] = jax.nn.sigmoid(h)


def _head(agg2, fc1p, fc1_b, ln1_g, ln1_b, fc2_w, fc2_b, ln2_g, ln2_b,
          fc3_w, fc3_b):
    return pl.pallas_call(
        _head_body,
        out_shape=jax.ShapeDtypeStruct((B, 1), jnp.float32),
    )(agg2, fc1p, fc1_b[None, :], ln1_g[None, :], ln1_b[None, :],
      fc2_w, fc2_b[None, :], ln2_g[None, :], ln2_b[None, :],
      fc3_w, fc3_b[None, :])


def kernel(x1, edge_index1, edge_attr1, batch1, x2, edge_index2, edge_attr2,
           W1, b1, W2, b2, fc1_w, fc1_b, ln1_g, ln1_b, fc2_w, fc2_b,
           ln2_g, ln2_b, fc3_w, fc3_b):
    pad_idx = (jnp.arange(EP - E, dtype=jnp.int32) % NCP)
    src = jnp.concatenate([edge_index1[0].astype(jnp.int32),
                           edge_index2[0].astype(jnp.int32) + N1, pad_idx])
    dst = jnp.concatenate([edge_index1[1].astype(jnp.int32),
                           edge_index2[1].astype(jnp.int32) + N1, pad_idx])
    w = jnp.concatenate([edge_attr1, edge_attr2,
                         jnp.zeros((EP - E,), jnp.float32)])

    xp = jnp.concatenate([x1, x2, jnp.zeros((NCP - NC, F), jnp.float32)])

    wbits = lax.bitcast_convert_type(w, jnp.int32)
    pdeg = jnp.stack([dst.reshape(-1, 128), wbits.reshape(-1, 128)], axis=1)
    degp = _make_deg_kernel()(pdeg, jnp.zeros((NCP,), jnp.float32))
    deg = degp[0] + degp[1] + 1.0
    dinv = deg ** -0.5
    dinv2 = dinv * dinv

    CH = 64
    packed = jnp.stack([src.reshape(-1, CH), dst.reshape(-1, CH),
                        wbits.reshape(-1, CH)], axis=1)

    def layer(h_in, W, b, f):
        h = h_in @ W
        g = dinv[:, None] * h
        p = _make_agg_kernel(f)(packed, g, jnp.zeros((NCP, f), jnp.float32))
        out = dinv[:, None] * (p[0] + p[1]) + dinv2[:, None] * h + b
        return jax.nn.relu(out)

    h1 = layer(xp, W1, b1, 128)
    h2 = layer(h1, W2, b2, 64)
    o1, o2 = h2[:N1], h2[N1:NC]
    o2p = jnp.concatenate([o2, jnp.zeros((N2P - N2, 64), jnp.float32)])
    blast = jnp.concatenate([o2[N2 - 1:N2], jnp.zeros((7, 64), jnp.float32)])
    vals = _vals(o1, blast)[:, 0]

    M = jnp.where(batch1[None, :] == jnp.arange(B)[:, None], vals[None, :], -jnp.inf)
    tv, ti = jax.lax.top_k(M, K)
    mask = jnp.isfinite(tv)
    o1sel = o1[ti.reshape(B * K)]
    rowmask = mask.reshape(B * K, 1).astype(jnp.float32)
    dsel = _dsel(o1sel, o2p, rowmask)
    agg2 = dsel.reshape(B, K * N2P)

    fc1p = jnp.concatenate(
        [fc1_w.reshape(K, N2, 128),
         jnp.zeros((K, N2P - N2, 128), jnp.float32)], axis=1
    ).reshape(K * N2P, 128)
    return _head(agg2, fc1p, fc1_b, ln1_g, ln1_b, fc2_w, fc2_b,
                 ln2_g, ln2_b, fc3_w, fc3_b)
